# baseline (speedup 1.0000x reference)
import functools

import jax
import jax.numpy as jnp
from jax import lax
from jax.experimental import pallas as pl
from jax.experimental.pallas import tpu as pltpu
from jax.experimental.pallas import tpu_sc as plsc

VOCAB = 1000000
EMBED_DIM = 64
NUM_CORES = 2
NUM_SUBCORES = 16
NUM_WORKERS = NUM_CORES * NUM_SUBCORES
NBLK = VOCAB // 128
PACK_ROWS = VOCAB // 2


def _pack_call(table_t, tail_p):
    mesh = plsc.VectorSubcoreMesh(core_axis_name="c", subcore_axis_name="s")
    n_pairs = (NBLK // NUM_WORKERS + 2) // 2

    @functools.partial(
        pl.kernel,
        mesh=mesh,
        compiler_params=pltpu.CompilerParams(use_tc_tiling_on_sc=True, needs_layout_passes=False),
        out_type=jax.ShapeDtypeStruct((PACK_ROWS, 128), jnp.float32),
        scratch_types=[
            pltpu.VMEM((64, 137), jnp.float32),
            pltpu.VMEM((64, 137), jnp.float32),
            pltpu.VMEM((64, 128), jnp.float32),
            pltpu.VMEM((64, 128), jnp.float32),
            pltpu.VMEM((32, 128), jnp.float32),
            pltpu.SemaphoreType.DMA((2,)),
            pltpu.SemaphoreType.DMA((2,)),
            pltpu.SemaphoreType.DMA,
        ],
    )
    def k(tt_hbm, tail_hbm, p_hbm, in0, in1, st0, st1, tbuf, sem_in, sem_out,
          sem_t):
        wid = lax.axis_index("s") * NUM_CORES + lax.axis_index("c")
        ins = (in0, in1)
        sts = (st0, st1)
        lane = lax.iota(jnp.int32, 16)

        def start_in(blk, par):
            pltpu.async_copy(
                tt_hbm.at[:, pl.ds(blk * 128, 128)],
                ins[par].at[:, pl.ds(0, 128)],
                sem_in.at[par],
            )

        def start_out(blk, par):
            pltpu.async_copy(
                sts[par], p_hbm.at[pl.ds(blk * 64, 64)], sem_out.at[par]
            )

        def wait(sem, par, buf):
            pltpu.make_async_copy(
                tt_hbm.at[:, pl.ds(0, 128)], buf, sem.at[par]
            ).wait()

        @pl.when(wid == 0)
        def _():
            pltpu.async_copy(tail_hbm, tbuf, sem_t).wait()
            pltpu.async_copy(tbuf, p_hbm.at[pl.ds(NBLK * 64, 32)], sem_t).wait()

        start_in(wid, 0)
        start_in(wid + NUM_WORKERS, 1)

        def transpose_block(src, dst):
            @plsc.parallel_loop(0, 64, step=1, unroll=8)
            def _(i):
                for g in range(8):
                    rvec = lane + (g % 4) * 16
                    cvec = jnp.full((16,), 2 * i + (1 if g >= 4 else 0),
                                    jnp.int32)
                    dst[i, pl.ds(g * 16, 16)] = plsc.load_gather(
                        src, [rvec, cvec]
                    )

        def pair(kk, carry):
            for par in range(2):
                kstep = kk * 2 + par
                blk = wid + kstep * NUM_WORKERS

                @pl.when(blk < NBLK)
                def _():
                    wait(sem_in, par, ins[par].at[:, pl.ds(0, 128)])

                    @pl.when(kstep >= 2)
                    def _():
                        wait(sem_out, par, sts[par])

                    transpose_block(ins[par], sts[par])
                    start_out(blk, par)
                    nblk = blk + 2 * NUM_WORKERS

                    @pl.when(nblk < NBLK)
                    def _():
                        start_in(nblk, par)

            return carry

        lax.fori_loop(0, n_pairs, pair, 0)

        for par in range(2):
            blk = wid + par * NUM_WORKERS

            @pl.when(blk < NBLK)
            def _():
                wait(sem_out, par, sts[par])

    return k(table_t, tail_p)


def _gather_call(idx_t, packed):
    n_ctx = idx_t.shape[0]
    batch = idx_t.shape[1]
    mesh = plsc.VectorSubcoreMesh(core_axis_name="c", subcore_axis_name="s")

    @functools.partial(
        pl.kernel,
        mesh=mesh,
        compiler_params=pltpu.CompilerParams(use_tc_tiling_on_sc=True, needs_layout_passes=False),
        out_type=jax.ShapeDtypeStruct((n_ctx, EMBED_DIM, batch), jnp.float32),
        scratch_types=[
            pltpu.VMEM((n_ctx, 128), jnp.int32),
            pltpu.VMEM((2, 128), jnp.int32),
            pltpu.VMEM((2, 128), jnp.int32),
            pltpu.VMEM((128, 128), jnp.float32),
            pltpu.VMEM((128, 128), jnp.float32),
            pltpu.VMEM((64, 128), jnp.float32),
            pltpu.VMEM((64, 128), jnp.float32),
            pltpu.SemaphoreType.DMA((2,)),
            pltpu.SemaphoreType.DMA((2,)),
            pltpu.SemaphoreType.DMA,
        ],
    )
    def k(idx_hbm, p_hbm, out_hbm, idxv, ridx, cbase, r0, r1, s0, s1,
          sem_g, sem_o, sem_i):
        wid = lax.axis_index("s") * NUM_CORES + lax.axis_index("c")
        b0 = wid * 128
        rows = (r0, r1)
        sts = (s0, s1)
        lane = lax.iota(jnp.int32, 16)

        pltpu.async_copy(idx_hbm.at[:, pl.ds(b0, 128)], idxv, sem_i).wait()

        def prep(t, par):
            for g in range(8):
                v = idxv[t, pl.ds(g * 16, 16)]
                ridx[par, pl.ds(g * 16, 16)] = v >> 1
                cbase[par, pl.ds(g * 16, 16)] = (v & 1) * 64

        def start_gather(par):
            pltpu.async_copy(
                p_hbm.at[ridx.at[par]], rows[par], sem_g.at[par]
            )

        def start_out(t, par):
            pltpu.async_copy(
                sts[par], out_hbm.at[t, :, pl.ds(b0, 128)], sem_o.at[par]
            )

        def wait(sem, par, buf):
            pltpu.make_async_copy(
                p_hbm.at[pl.ds(0, buf.shape[0])], buf, sem.at[par]
            ).wait()

        def extract(t, par):
            cb = [cbase[par, pl.ds(g * 16, 16)] for g in range(8)]

            @plsc.parallel_loop(0, EMBED_DIM, step=1, unroll=8)
            def _(d):
                for g in range(8):
                    rvec = lane + g * 16
                    sts[par][d, pl.ds(g * 16, 16)] = plsc.load_gather(
                        rows[par], [rvec, cb[g] + d]
                    )

        prep(0, 0)
        start_gather(0)

        def pair(kk, carry):
            for par in range(2):
                t = kk * 2 + par

                @pl.when(t < n_ctx)
                def _():
                    tn = t + 1

                    @pl.when(tn < n_ctx)
                    def _():
                        prep(tn, 1 - par)
                        start_gather(1 - par)

                    wait(sem_g, par, rows[par])

                    @pl.when(t >= 2)
                    def _():
                        wait(sem_o, par, sts[par])

                    extract(t, par)
                    start_out(t, par)

            return carry

        lax.fori_loop(0, (n_ctx + 1) // 2, pair, 0)

        for par in range(2):
            @pl.when(par < n_ctx)
            def _():
                wait(sem_o, par, sts[par])

    return k(idx_t, packed)


def kernel(indices, target_table):
    n_batch, n_ctx = indices.shape
    idx_t = indices.T
    table_t = target_table.T
    tail_p = target_table[NBLK * 128:].reshape(32, 128)
    packed = _pack_call(table_t, tail_p)
    out_t = _gather_call(idx_t, packed)
    return out_t.transpose(2, 0, 1)


# speedup vs baseline: 1.2594x; 1.2594x over previous
import functools

import jax
import jax.numpy as jnp
from jax import lax
from jax.experimental import pallas as pl
from jax.experimental.pallas import tpu as pltpu
from jax.experimental.pallas import tpu_sc as plsc

VOCAB = 1000000
EMBED_DIM = 64
NUM_CORES = 2
NUM_SUBCORES = 16
NUM_WORKERS = NUM_CORES * NUM_SUBCORES
NBLK = VOCAB // 128
PACK_ROWS = VOCAB // 2


def _pack_call(table_t, tail_p):
    mesh = plsc.VectorSubcoreMesh(core_axis_name="c", subcore_axis_name="s")
    n_pairs = (NBLK // NUM_WORKERS + 2) // 2

    @functools.partial(
        pl.kernel,
        mesh=mesh,
        compiler_params=pltpu.CompilerParams(use_tc_tiling_on_sc=True, needs_layout_passes=False),
        out_type=jax.ShapeDtypeStruct((PACK_ROWS, 128), jnp.float32),
        scratch_types=[
            pltpu.VMEM((64, 137), jnp.float32),
            pltpu.VMEM((64, 137), jnp.float32),
            pltpu.VMEM((64, 128), jnp.float32),
            pltpu.VMEM((64, 128), jnp.float32),
            pltpu.VMEM((32, 128), jnp.float32),
            pltpu.SemaphoreType.DMA((2,)),
            pltpu.SemaphoreType.DMA((2,)),
            pltpu.SemaphoreType.DMA,
        ],
    )
    def k(tt_hbm, tail_hbm, p_hbm, in0, in1, st0, st1, tbuf, sem_in, sem_out,
          sem_t):
        wid = lax.axis_index("s") * NUM_CORES + lax.axis_index("c")
        ins = (in0, in1)
        sts = (st0, st1)
        lane = lax.iota(jnp.int32, 16)

        def start_in(blk, par):
            pltpu.async_copy(
                tt_hbm.at[:, pl.ds(blk * 128, 128)],
                ins[par].at[:, pl.ds(0, 128)],
                sem_in.at[par],
            )

        def start_out(blk, par):
            pltpu.async_copy(
                sts[par], p_hbm.at[pl.ds(blk * 64, 64)], sem_out.at[par]
            )

        def wait(sem, par, buf):
            pltpu.make_async_copy(
                tt_hbm.at[:, pl.ds(0, 128)], buf, sem.at[par]
            ).wait()

        @pl.when(wid == 0)
        def _():
            pltpu.async_copy(tail_hbm, tbuf, sem_t).wait()
            pltpu.async_copy(tbuf, p_hbm.at[pl.ds(NBLK * 64, 32)], sem_t).wait()

        start_in(wid, 0)
        start_in(wid + NUM_WORKERS, 1)

        def transpose_block(src, dst):
            @plsc.parallel_loop(0, 64, step=1, unroll=8)
            def _(i):
                for g in range(8):
                    rvec = lane + (g % 4) * 16
                    cvec = jnp.full((16,), 2 * i + (1 if g >= 4 else 0),
                                    jnp.int32)
                    dst[i, pl.ds(g * 16, 16)] = plsc.load_gather(
                        src, [rvec, cvec]
                    )

        def pair(kk, carry):
            for par in range(2):
                kstep = kk * 2 + par
                blk = wid + kstep * NUM_WORKERS

                @pl.when(blk < NBLK)
                def _():
                    wait(sem_in, par, ins[par].at[:, pl.ds(0, 128)])

                    @pl.when(kstep >= 2)
                    def _():
                        wait(sem_out, par, sts[par])

                    transpose_block(ins[par], sts[par])
                    start_out(blk, par)
                    nblk = blk + 2 * NUM_WORKERS

                    @pl.when(nblk < NBLK)
                    def _():
                        start_in(nblk, par)

            return carry

        lax.fori_loop(0, n_pairs, pair, 0)

        for par in range(2):
            blk = wid + par * NUM_WORKERS

            @pl.when(blk < NBLK)
            def _():
                wait(sem_out, par, sts[par])

    return k(table_t, tail_p)


def _gather_call(idx_t, packed):
    n_ctx = idx_t.shape[0]
    batch = idx_t.shape[1]
    mesh = plsc.VectorSubcoreMesh(core_axis_name="c", subcore_axis_name="s")

    @functools.partial(
        pl.kernel,
        mesh=mesh,
        compiler_params=pltpu.CompilerParams(use_tc_tiling_on_sc=True, needs_layout_passes=False),
        out_type=jax.ShapeDtypeStruct((n_ctx, EMBED_DIM, batch), jnp.float32),
        scratch_types=[
            pltpu.VMEM((n_ctx, 128), jnp.int32),
            pltpu.VMEM((2, 128), jnp.int32),
            pltpu.VMEM((2, 128), jnp.int32),
            pltpu.VMEM((128, 128), jnp.float32),
            pltpu.VMEM((128, 128), jnp.float32),
            pltpu.VMEM((64, 128), jnp.float32),
            pltpu.VMEM((64, 128), jnp.float32),
            pltpu.SemaphoreType.DMA((2,)),
            pltpu.SemaphoreType.DMA((2,)),
            pltpu.SemaphoreType.DMA,
        ],
    )
    def k(idx_hbm, p_hbm, out_hbm, idxv, ridx, cbase, r0, r1, s0, s1,
          sem_g, sem_o, sem_i):
        wid = lax.axis_index("s") * NUM_CORES + lax.axis_index("c")
        b0 = wid * 128
        rows = (r0, r1)
        sts = (s0, s1)
        lane = lax.iota(jnp.int32, 16)

        pltpu.async_copy(idx_hbm.at[:, pl.ds(b0, 128)], idxv, sem_i).wait()

        def prep(t, par):
            for g in range(8):
                v = idxv[t, pl.ds(g * 16, 16)]
                ridx[par, pl.ds(g * 16, 16)] = v >> 1
                cbase[par, pl.ds(g * 16, 16)] = (v & 1) * 64

        def start_gather(par):
            pltpu.async_copy(
                p_hbm.at[ridx.at[par]], rows[par], sem_g.at[par]
            )

        def start_out(t, par):
            pltpu.async_copy(
                sts[par], out_hbm.at[t, :, pl.ds(b0, 128)], sem_o.at[par]
            )

        def wait(sem, par, buf):
            pltpu.make_async_copy(
                p_hbm.at[pl.ds(0, buf.shape[0])], buf, sem.at[par]
            ).wait()

        def extract(t, par):
            cb = [cbase[par, pl.ds(g * 16, 16)] for g in range(8)]

            @plsc.parallel_loop(0, EMBED_DIM, step=1, unroll=8)
            def _(d):
                for g in range(8):
                    rvec = lane + g * 16
                    sts[par][d, pl.ds(g * 16, 16)] = plsc.load_gather(
                        rows[par], [rvec, cb[g] + d]
                    )

        prep(0, 0)
        start_gather(0)

        def pair(kk, carry):
            for par in range(2):
                t = kk * 2 + par

                @pl.when(t < n_ctx)
                def _():
                    tn = t + 1

                    @pl.when(tn < n_ctx)
                    def _():
                        prep(tn, 1 - par)
                        start_gather(1 - par)

                    wait(sem_g, par, rows[par])

                    @pl.when(t >= 2)
                    def _():
                        wait(sem_o, par, sts[par])

                    extract(t, par)
                    start_out(t, par)

            return carry

        lax.fori_loop(0, (n_ctx + 1) // 2, pair, 0)

        for par in range(2):
            @pl.when(par < n_ctx)
            def _():
                wait(sem_o, par, sts[par])

    return k(idx_t, packed)


def kernel(indices, target_table):
    n_batch, n_ctx = indices.shape
    idx_t = indices.T
    packed = target_table.reshape(PACK_ROWS, 128)
    out_t = _gather_call(idx_t, packed)
    return out_t.transpose(2, 0, 1)


# speedup vs baseline: 1.3752x; 1.0919x over previous
import functools

import jax
import jax.numpy as jnp
from jax import lax
from jax.experimental import pallas as pl
from jax.experimental.pallas import tpu as pltpu
from jax.experimental.pallas import tpu_sc as plsc

VOCAB = 1000000
EMBED_DIM = 64
NUM_CORES = 2
NUM_SUBCORES = 16
NUM_WORKERS = NUM_CORES * NUM_SUBCORES
NBLK = VOCAB // 128
PACK_ROWS = VOCAB // 2


def _pack_call(table_t, tail_p):
    mesh = plsc.VectorSubcoreMesh(core_axis_name="c", subcore_axis_name="s")
    n_pairs = (NBLK // NUM_WORKERS + 2) // 2

    @functools.partial(
        pl.kernel,
        mesh=mesh,
        compiler_params=pltpu.CompilerParams(use_tc_tiling_on_sc=True, needs_layout_passes=False),
        out_type=jax.ShapeDtypeStruct((PACK_ROWS, 128), jnp.float32),
        scratch_types=[
            pltpu.VMEM((64, 137), jnp.float32),
            pltpu.VMEM((64, 137), jnp.float32),
            pltpu.VMEM((64, 128), jnp.float32),
            pltpu.VMEM((64, 128), jnp.float32),
            pltpu.VMEM((32, 128), jnp.float32),
            pltpu.SemaphoreType.DMA((2,)),
            pltpu.SemaphoreType.DMA((2,)),
            pltpu.SemaphoreType.DMA,
        ],
    )
    def k(tt_hbm, tail_hbm, p_hbm, in0, in1, st0, st1, tbuf, sem_in, sem_out,
          sem_t):
        wid = lax.axis_index("s") * NUM_CORES + lax.axis_index("c")
        ins = (in0, in1)
        sts = (st0, st1)
        lane = lax.iota(jnp.int32, 16)

        def start_in(blk, par):
            pltpu.async_copy(
                tt_hbm.at[:, pl.ds(blk * 128, 128)],
                ins[par].at[:, pl.ds(0, 128)],
                sem_in.at[par],
            )

        def start_out(blk, par):
            pltpu.async_copy(
                sts[par], p_hbm.at[pl.ds(blk * 64, 64)], sem_out.at[par]
            )

        def wait(sem, par, buf):
            pltpu.make_async_copy(
                tt_hbm.at[:, pl.ds(0, 128)], buf, sem.at[par]
            ).wait()

        @pl.when(wid == 0)
        def _():
            pltpu.async_copy(tail_hbm, tbuf, sem_t).wait()
            pltpu.async_copy(tbuf, p_hbm.at[pl.ds(NBLK * 64, 32)], sem_t).wait()

        start_in(wid, 0)
        start_in(wid + NUM_WORKERS, 1)

        def transpose_block(src, dst):
            @plsc.parallel_loop(0, 64, step=1, unroll=8)
            def _(i):
                for g in range(8):
                    rvec = lane + (g % 4) * 16
                    cvec = jnp.full((16,), 2 * i + (1 if g >= 4 else 0),
                                    jnp.int32)
                    dst[i, pl.ds(g * 16, 16)] = plsc.load_gather(
                        src, [rvec, cvec]
                    )

        def pair(kk, carry):
            for par in range(2):
                kstep = kk * 2 + par
                blk = wid + kstep * NUM_WORKERS

                @pl.when(blk < NBLK)
                def _():
                    wait(sem_in, par, ins[par].at[:, pl.ds(0, 128)])

                    @pl.when(kstep >= 2)
                    def _():
                        wait(sem_out, par, sts[par])

                    transpose_block(ins[par], sts[par])
                    start_out(blk, par)
                    nblk = blk + 2 * NUM_WORKERS

                    @pl.when(nblk < NBLK)
                    def _():
                        start_in(nblk, par)

            return carry

        lax.fori_loop(0, n_pairs, pair, 0)

        for par in range(2):
            blk = wid + par * NUM_WORKERS

            @pl.when(blk < NBLK)
            def _():
                wait(sem_out, par, sts[par])

    return k(table_t, tail_p)


def _gather_call(idx_t, packed):
    n_ctx = idx_t.shape[0]
    batch = idx_t.shape[1]
    mesh = plsc.VectorSubcoreMesh(core_axis_name="c", subcore_axis_name="s")

    @functools.partial(
        pl.kernel,
        mesh=mesh,
        compiler_params=pltpu.CompilerParams(use_tc_tiling_on_sc=True, needs_layout_passes=False),
        out_type=jax.ShapeDtypeStruct((n_ctx, EMBED_DIM, batch), jnp.float32),
        scratch_types=[
            pltpu.VMEM((n_ctx, 128), jnp.int32),
            pltpu.VMEM((2, 128), jnp.int32),
            pltpu.VMEM((2, 128), jnp.int32),
            pltpu.VMEM((128, 128), jnp.float32),
            pltpu.VMEM((128, 128), jnp.float32),
            pltpu.VMEM((64, 128), jnp.float32),
            pltpu.VMEM((64, 128), jnp.float32),
            pltpu.SemaphoreType.DMA((2,)),
            pltpu.SemaphoreType.DMA((2,)),
            pltpu.SemaphoreType.DMA,
        ],
    )
    def k(idx_hbm, p_hbm, out_hbm, idxv, ridx, cbase, r0, r1, s0, s1,
          sem_g, sem_o, sem_i):
        wid = lax.axis_index("s") * NUM_CORES + lax.axis_index("c")
        b0 = wid * 128
        rows = (r0, r1)
        sts = (s0, s1)
        lane = lax.iota(jnp.int32, 16)

        pltpu.async_copy(idx_hbm.at[:, pl.ds(b0, 128)], idxv, sem_i).wait()

        half = packed.shape[1] // EMBED_DIM

        def prep(t, par):
            for g in range(8):
                v = idxv[t, pl.ds(g * 16, 16)]
                if half == 2:
                    ridx[par, pl.ds(g * 16, 16)] = v >> 1
                    cbase[par, pl.ds(g * 16, 16)] = (v & 1) * 64
                else:
                    ridx[par, pl.ds(g * 16, 16)] = v
                    cbase[par, pl.ds(g * 16, 16)] = v - v

        def start_gather(par):
            pltpu.async_copy(
                p_hbm.at[ridx.at[par]], rows[par], sem_g.at[par]
            )

        def start_out(t, par):
            pltpu.async_copy(
                sts[par], out_hbm.at[t, :, pl.ds(b0, 128)], sem_o.at[par]
            )

        def wait(sem, par, buf):
            pltpu.make_async_copy(
                p_hbm.at[pl.ds(0, buf.shape[0])], buf, sem.at[par]
            ).wait()

        def extract(t, par):
            cb = [cbase[par, pl.ds(g * 16, 16)] for g in range(8)]

            @plsc.parallel_loop(0, EMBED_DIM, step=1, unroll=8)
            def _(d):
                for g in range(8):
                    rvec = lane + g * 16
                    sts[par][d, pl.ds(g * 16, 16)] = plsc.load_gather(
                        rows[par], [rvec, cb[g] + d]
                    )

        prep(0, 0)
        start_gather(0)

        def pair(kk, carry):
            for par in range(2):
                t = kk * 2 + par

                @pl.when(t < n_ctx)
                def _():
                    tn = t + 1

                    @pl.when(tn < n_ctx)
                    def _():
                        prep(tn, 1 - par)
                        start_gather(1 - par)

                    wait(sem_g, par, rows[par])

                    @pl.when(t >= 2)
                    def _():
                        wait(sem_o, par, sts[par])

                    extract(t, par)
                    start_out(t, par)

            return carry

        lax.fori_loop(0, (n_ctx + 1) // 2, pair, 0)

        for par in range(2):
            @pl.when(par < n_ctx)
            def _():
                wait(sem_o, par, sts[par])

    return k(idx_t, packed)


def kernel(indices, target_table):
    n_batch, n_ctx = indices.shape
    idx_t = indices.T
    padded = jnp.pad(target_table, ((0, 0), (0, 128 - EMBED_DIM)))
    out_t = _gather_call(idx_t, padded)
    return out_t.transpose(2, 0, 1)
